# Initial kernel scaffold; baseline (speedup 1.0000x reference)
import jax
import jax.numpy as jnp
from jax.experimental import pallas as pl


def kernel(mask, weight):
    raise NotImplementedError("write your pallas kernel here")


# speedup vs baseline: 1.5729x; 1.5729x over previous
import functools

import jax
import jax.numpy as jnp
from jax import lax
from jax.experimental import pallas as pl
from jax.experimental.pallas import tpu as pltpu
from jax.experimental.pallas import tpu_sc as plsc


def _make_gather(vocab: int, emb: int, total: int):
    info = plsc.get_sparse_core_info()
    nc, ns = info.num_cores, info.num_subcores
    nw = nc * ns
    C = 128
    per_w = total // nw
    n_chunks = per_w // C
    assert total % (nw * C) == 0

    mesh = plsc.VectorSubcoreMesh(core_axis_name="c", subcore_axis_name="s")

    @functools.partial(
        pl.kernel,
        mesh=mesh,
        out_type=jax.ShapeDtypeStruct((total, emb), jnp.float32),
        scratch_types=[
            pltpu.VMEM((C,), jnp.int32),
            pltpu.VMEM((C, emb), jnp.float32),
            pltpu.SemaphoreType.DMA,
        ],
        compiler_params=pltpu.CompilerParams(use_tc_tiling_on_sc=False),
    )
    def gather_kernel(idx_hbm, table_hbm, out_hbm, idx_v, rows_v, sem):
        wid = lax.axis_index("s") * nc + lax.axis_index("c")
        base = wid * per_w

        def chunk_body(i, carry):
            off = pl.multiple_of(base + i * C, C)
            pltpu.sync_copy(idx_hbm.at[pl.ds(off, C)], idx_v)
            pltpu.async_copy(table_hbm.at[idx_v], rows_v, sem).wait()
            pltpu.sync_copy(rows_v, out_hbm.at[pl.ds(off, C)])
            return carry

        lax.fori_loop(0, n_chunks, chunk_body, 0)

    return gather_kernel


def kernel(mask, weight):
    b, l = mask.shape
    vocab, emb = weight.shape
    total = b * l
    idx = mask.reshape(total)
    out = _make_gather(vocab, emb, total)(idx, weight)
    return out.reshape(b, l, emb)


# speedup vs baseline: 1.8663x; 1.1865x over previous
import functools

import jax
import jax.numpy as jnp
from jax import lax
from jax.experimental import pallas as pl
from jax.experimental.pallas import tpu as pltpu
from jax.experimental.pallas import tpu_sc as plsc


def _make_gather(vocab: int, emb: int, total: int):
    info = plsc.get_sparse_core_info()
    nc, ns = info.num_cores, info.num_subcores
    nw = nc * ns
    C = 128
    K = 4
    S = K * C
    per_w = total // nw
    n_super = per_w // S
    assert total % (nw * S) == 0

    mesh = plsc.VectorSubcoreMesh(core_axis_name="c", subcore_axis_name="s")

    @functools.partial(
        pl.kernel,
        mesh=mesh,
        out_type=jax.ShapeDtypeStruct((total, emb), jnp.float32),
        scratch_types=[
            pltpu.VMEM((2, S), jnp.int32),
            pltpu.VMEM((2, S, emb), jnp.float32),
            pltpu.SemaphoreType.DMA,
            pltpu.SemaphoreType.DMA,
            pltpu.SemaphoreType.DMA,
        ],
        compiler_params=pltpu.CompilerParams(use_tc_tiling_on_sc=False),
    )
    def gather_kernel(idx_hbm, table_hbm, out_hbm, idx_v, rows_v, isem, gsem, osem):
        wid = lax.axis_index("s") * nc + lax.axis_index("c")
        base = wid * per_w

        pltpu.make_async_copy(
            idx_hbm.at[pl.ds(base, S)], idx_v.at[0], isem).start()

        def super_body(g, carry):
            s = lax.rem(g, 2)
            off = pl.multiple_of(base + g * S, S)

            @pl.when(g >= 2)
            def _():
                pltpu.make_async_copy(
                    rows_v.at[s], out_hbm.at[pl.ds(off, S)], osem).wait()

            pltpu.make_async_copy(
                idx_hbm.at[pl.ds(off, S)], idx_v.at[s], isem).wait()

            for k in range(K):
                pltpu.make_async_copy(
                    table_hbm.at[idx_v.at[s, pl.ds(k * C, C)]],
                    rows_v.at[s, pl.ds(k * C, C)], gsem).start()

            @pl.when(g + 1 < n_super)
            def _():
                pltpu.make_async_copy(
                    idx_hbm.at[pl.ds(off + S, S)], idx_v.at[1 - s], isem).start()

            for k in range(K):
                pltpu.make_async_copy(
                    table_hbm.at[idx_v.at[s, pl.ds(k * C, C)]],
                    rows_v.at[s, pl.ds(k * C, C)], gsem).wait()
            pltpu.make_async_copy(
                rows_v.at[s], out_hbm.at[pl.ds(off, S)], osem).start()
            return carry

        lax.fori_loop(0, n_super, super_body, 0)

        for t in (n_super - 2, n_super - 1):
            s = t % 2
            off = base + t * S
            pltpu.make_async_copy(
                rows_v.at[s], out_hbm.at[pl.ds(off, S)], osem).wait()

    return gather_kernel


def kernel(mask, weight):
    b, l = mask.shape
    vocab, emb = weight.shape
    total = b * l
    idx = mask.reshape(total)
    out = _make_gather(vocab, emb, total)(idx, weight)
    return out.reshape(b, l, emb)
